# baseline (speedup 1.0000x reference)
import functools
import math

import jax
import jax.numpy as jnp
from jax import lax
from jax.experimental import pallas as pl
from jax.experimental.pallas import tpu as pltpu
from jax.experimental.pallas import tpu_sc as plsc

D = 64
B = 16384
SCALE = 1.0 / math.sqrt(4.0)

NC = 2
NS = 16
NW = NC * NS
BPW = B // NW
CH = 128
NCHUNK = BPW // CH

SLAB = 128

V_U, V_I, V_C = 1000001, 100001, 1001


def _geom(v):
    full = v // SLAB
    tail = v - full * SLAB
    h = (v + 1) // 2
    return full, tail, h


U_FULL, U_TAIL, H_U = _geom(V_U)
I_FULL, I_TAIL, H_I = _geom(V_I)
C_FULL, C_TAIL, H_C = _geom(V_C)


def _adj_body(dt_ref, w_ref, o_ref):
    acc = lax.dot_general(dt_ref[...], w_ref[...], (((0,), (0,)), ((), ())),
                          preferred_element_type=jnp.float32)
    o_ref[...] = acc * SCALE


def _adjustment(dense_t, W_dense):
    return pl.pallas_call(
        _adj_body,
        out_shape=jax.ShapeDtypeStruct((B, D), jnp.float32),
    )(dense_t, W_dense)


def _shuffle_slab(slab_v, out_v, nq):
    def qloop(q, carry):
        col0 = jnp.full((16,), 2 * q, jnp.int32)
        col1 = col0 + 1
        for c4 in range(D // 16):
            rows = c4 * 16 + lax.iota(jnp.int32, 16)
            out_v[q, pl.ds(c4 * 16, 16)] = plsc.load_gather(
                slab_v, [rows, col0])
            out_v[q, pl.ds(64 + c4 * 16, 16)] = plsc.load_gather(
                slab_v, [rows, col1])
        return carry

    lax.fori_loop(0, nq, qloop, 0)


def _repack_body(eut_hbm, eit_hbm, ect_hbm, pu_hbm, pi_hbm, pc_hbm,
                 slab_v, out_v):
    wid = lax.axis_index("s") * NC + lax.axis_index("c")

    def do_table(et_hbm, p_hbm, nfull):
        nk = -(-nfull // NW)

        def loop(k, carry):
            j = wid + k * NW

            @pl.when(j < nfull)
            def _():
                pltpu.sync_copy(et_hbm.at[:, pl.ds(j * SLAB, SLAB)], slab_v)
                _shuffle_slab(slab_v, out_v, SLAB // 2)
                pltpu.sync_copy(out_v, p_hbm.at[pl.ds(j * (SLAB // 2),
                                                      SLAB // 2)])
            return carry

        lax.fori_loop(0, nk, loop, 0)

    do_table(eut_hbm, pu_hbm, U_FULL)
    do_table(eit_hbm, pi_hbm, I_FULL)
    do_table(ect_hbm, pc_hbm, C_FULL)


_repack_call = functools.partial(
    pl.kernel,
    out_type=(
        jax.ShapeDtypeStruct((H_U, 128), jnp.float32),
        jax.ShapeDtypeStruct((H_I, 128), jnp.float32),
        jax.ShapeDtypeStruct((H_C, 128), jnp.float32),
    ),
    mesh=plsc.VectorSubcoreMesh(core_axis_name="c", subcore_axis_name="s"),
    scratch_types=[
        pltpu.VMEM((D, SLAB), jnp.float32),
        pltpu.VMEM((SLAB // 2, 128), jnp.float32),
    ],
    compiler_params=pltpu.CompilerParams(needs_layout_passes=False),
)(_repack_body)


def _tail_body(et_ref, p_ref, o_ref):
    y = et_ref[...].T
    q = lax.broadcasted_iota(jnp.int32, (SLAB // 2, SLAB), 0)
    j = lax.broadcasted_iota(jnp.int32, (SLAB // 2, SLAB), 1)
    p0 = (j == 2 * q).astype(jnp.float32)
    p1 = (j == 2 * q + 1).astype(jnp.float32)
    ev = lax.dot_general(p0, y, (((1,), (0,)), ((), ())),
                         preferred_element_type=jnp.float32,
                         precision=lax.Precision.HIGHEST)
    od = lax.dot_general(p1, y, (((1,), (0,)), ((), ())),
                         preferred_element_type=jnp.float32,
                         precision=lax.Precision.HIGHEST)
    o_ref[...] = jnp.concatenate([ev, od], axis=1)


def _tail_fix(et, p, full_blk):
    h = p.shape[0]
    return pl.pallas_call(
        _tail_body,
        grid=(1,),
        in_specs=[
            pl.BlockSpec((D, SLAB), lambda q, f=full_blk: (0, f)),
            pl.BlockSpec((8, 128), lambda q: (0, 0)),
        ],
        out_specs=pl.BlockSpec((SLAB // 2, 128),
                               lambda q, f=full_blk: (f, 0)),
        out_shape=jax.ShapeDtypeStruct((h, 128), jnp.float32),
        input_output_aliases={1: 0},
    )(et, p)


def _sc_body(uid_hbm, iid_hbm, cid_hbm, adj_hbm, pu_hbm, pi_hbm, pc_hbm,
             out_hbm, idxu_v, idxi_v, idxc_v, mapu_v, mapi_v, mapc_v,
             ru_v, ri_v, rc_v, adj_v, sem, adj_sem):
    wid = lax.axis_index("s") * NC + lax.axis_index("c")
    base = wid * BPW

    def chunk(ci, carry):
        off = base + ci * CH
        pltpu.sync_copy(uid_hbm.at[pl.ds(off, CH)], idxu_v)
        pltpu.sync_copy(iid_hbm.at[pl.ds(off, CH)], idxi_v)
        pltpu.sync_copy(cid_hbm.at[pl.ds(off, CH)], idxc_v)
        ca = pltpu.async_copy(adj_hbm.at[pl.ds(off, CH)], adj_v, adj_sem)

        def remap(g, rcarry):
            s = pl.ds(g * 16, 16)
            mapu_v[s] = lax.shift_right_logical(idxu_v[s], 1)
            mapi_v[s] = lax.shift_right_logical(idxi_v[s], 1)
            mapc_v[s] = lax.shift_right_logical(idxc_v[s], 1)
            return rcarry

        lax.fori_loop(0, CH // 16, remap, 0)
        cu = pltpu.async_copy(pu_hbm.at[mapu_v], ru_v, sem)
        cit = pltpu.async_copy(pi_hbm.at[mapi_v], ri_v, sem)
        cc = pltpu.async_copy(pc_hbm.at[mapc_v], rc_v, sem)
        cu.wait()
        cit.wait()
        cc.wait()
        ca.wait()

        def combine(g, rcarry):
            sl = pl.ds(g * 16, 16)
            vu = idxu_v[sl]
            vi = idxi_v[sl]
            vc = idxc_v[sl]
            for l in range(16):
                r = g * 16 + l
                su = jnp.where(vu[l] == 0, 0.0, SCALE)
                si = jnp.where(vi[l] == 0, 0.0, SCALE)
                sc = jnp.where(vc[l] == 0, 0.0, SCALE)
                hu = (vu[l] & 1) == 1
                hi_ = (vi[l] & 1) == 1
                hc = (vc[l] & 1) == 1
                for c4 in range(D // 16):
                    c16 = c4 * 16
                    s = pl.ds(c16, 16)
                    s2 = pl.ds(64 + c16, 16)
                    gu = jnp.where(hu, ru_v[r, s2], ru_v[r, s])
                    gi = jnp.where(hi_, ri_v[r, s2], ri_v[r, s])
                    gc = jnp.where(hc, rc_v[r, s2], rc_v[r, s])
                    adj_v[r, s] = (gu * su + gi * si + gc * sc
                                   + adj_v[r, s])
            return rcarry

        lax.fori_loop(0, CH // 16, combine, 0)
        pltpu.sync_copy(adj_v, out_hbm.at[pl.ds(off, CH)])
        return carry

    lax.fori_loop(0, NCHUNK, chunk, 0)


_sc_call = functools.partial(
    pl.kernel,
    out_type=jax.ShapeDtypeStruct((B, D), jnp.float32),
    mesh=plsc.VectorSubcoreMesh(core_axis_name="c", subcore_axis_name="s"),
    scratch_types=[
        pltpu.VMEM((CH,), jnp.int32),
        pltpu.VMEM((CH,), jnp.int32),
        pltpu.VMEM((CH,), jnp.int32),
        pltpu.VMEM((CH,), jnp.int32),
        pltpu.VMEM((CH,), jnp.int32),
        pltpu.VMEM((CH,), jnp.int32),
        pltpu.VMEM((CH, 128), jnp.float32),
        pltpu.VMEM((CH, 128), jnp.float32),
        pltpu.VMEM((CH, 128), jnp.float32),
        pltpu.VMEM((CH, D), jnp.float32),
        pltpu.SemaphoreType.DMA,
        pltpu.SemaphoreType.DMA,
    ],
)(_sc_body)


def kernel(user_id, item_id, category, dense_0, E_user, E_item, E_cat,
           W_dense):
    u = user_id.astype(jnp.int32)
    i = item_id.astype(jnp.int32)
    c = category.astype(jnp.int32)
    pu, pi, pc = _repack_call(E_user.T, E_item.T, E_cat.T)
    pu = _tail_fix(E_user.T, pu, U_FULL)
    pi = _tail_fix(E_item.T, pi, I_FULL)
    pc = _tail_fix(E_cat.T, pc, C_FULL)
    adj = _adjustment(dense_0.T, W_dense)
    return _sc_call(u, i, c, adj, pu, pi, pc)


# speedup vs baseline: 2.4113x; 2.4113x over previous
import functools
import math

import jax
import jax.numpy as jnp
from jax import lax
from jax.experimental import pallas as pl
from jax.experimental.pallas import tpu as pltpu
from jax.experimental.pallas import tpu_sc as plsc

D = 64
B = 16384
SCALE = 1.0 / math.sqrt(4.0)

NC = 2
NS = 16
NW = NC * NS
BPW = B // NW
CH = 128
NCHUNK = BPW // CH


def _adj_body(dt_ref, w_ref, o_ref):
    acc = lax.dot_general(dt_ref[...], w_ref[...], (((0,), (0,)), ((), ())),
                          preferred_element_type=jnp.float32)
    o_ref[...] = acc * SCALE


def _adjustment(dense_t, W_dense):
    return pl.pallas_call(
        _adj_body,
        out_shape=jax.ShapeDtypeStruct((B, D), jnp.float32),
    )(dense_t, W_dense)


def _sc_body(uid_hbm, iid_hbm, cid_hbm, adj_hbm, pu_hbm, pi_hbm, pc_hbm,
             out_hbm, idxu_v, idxi_v, idxc_v, mapu_v, mapi_v, mapc_v,
             ru_v, ri_v, rc_v, adj_v, sem, adj_sem):
    wid = lax.axis_index("s") * NC + lax.axis_index("c")
    base = wid * BPW

    def chunk(ci, carry):
        off = base + ci * CH
        pltpu.sync_copy(uid_hbm.at[pl.ds(off, CH)], idxu_v)
        pltpu.sync_copy(iid_hbm.at[pl.ds(off, CH)], idxi_v)
        pltpu.sync_copy(cid_hbm.at[pl.ds(off, CH)], idxc_v)
        ca = pltpu.async_copy(adj_hbm.at[pl.ds(off, CH)], adj_v, adj_sem)

        def remap(g, rcarry):
            s = pl.ds(g * 16, 16)
            mapu_v[s] = lax.shift_right_logical(idxu_v[s], 1)
            mapi_v[s] = lax.shift_right_logical(idxi_v[s], 1)
            mapc_v[s] = lax.shift_right_logical(idxc_v[s], 1)
            return rcarry

        lax.fori_loop(0, CH // 16, remap, 0)
        cu = pltpu.async_copy(pu_hbm.at[mapu_v], ru_v, sem)
        cit = pltpu.async_copy(pi_hbm.at[mapi_v], ri_v, sem)
        cc = pltpu.async_copy(pc_hbm.at[mapc_v], rc_v, sem)
        cu.wait()
        cit.wait()
        cc.wait()
        ca.wait()

        def combine(g, rcarry):
            sl = pl.ds(g * 16, 16)
            vu = idxu_v[sl]
            vi = idxi_v[sl]
            vc = idxc_v[sl]
            for l in range(16):
                r = g * 16 + l
                hu = (vu[l] & 1) == 1
                hi_ = (vi[l] & 1) == 1
                hc = (vc[l] & 1) == 1
                for c4 in range(D // 16):
                    c16 = c4 * 16
                    s = pl.ds(c16, 16)
                    s2 = pl.ds(64 + c16, 16)
                    gu = jnp.where(hu, ru_v[r, s2], ru_v[r, s])
                    gi = jnp.where(hi_, ri_v[r, s2], ri_v[r, s])
                    gc = jnp.where(hc, rc_v[r, s2], rc_v[r, s])
                    adj_v[r, s] = (gu + gi + gc) * SCALE + adj_v[r, s]
            return rcarry

        lax.fori_loop(0, CH // 16, combine, 0)
        pltpu.sync_copy(adj_v, out_hbm.at[pl.ds(off, CH)])
        return carry

    lax.fori_loop(0, NCHUNK, chunk, 0)


_sc_call = functools.partial(
    pl.kernel,
    out_type=jax.ShapeDtypeStruct((B, D), jnp.float32),
    mesh=plsc.VectorSubcoreMesh(core_axis_name="c", subcore_axis_name="s"),
    scratch_types=[
        pltpu.VMEM((CH,), jnp.int32),
        pltpu.VMEM((CH,), jnp.int32),
        pltpu.VMEM((CH,), jnp.int32),
        pltpu.VMEM((CH,), jnp.int32),
        pltpu.VMEM((CH,), jnp.int32),
        pltpu.VMEM((CH,), jnp.int32),
        pltpu.VMEM((CH, 128), jnp.float32),
        pltpu.VMEM((CH, 128), jnp.float32),
        pltpu.VMEM((CH, 128), jnp.float32),
        pltpu.VMEM((CH, D), jnp.float32),
        pltpu.SemaphoreType.DMA,
        pltpu.SemaphoreType.DMA,
    ],
)(_sc_body)


def _pack(table, vocab):
    half = vocab // 2
    return jnp.reshape(table.at[0].set(0.0)[:2 * half], (half, 128))


def kernel(user_id, item_id, category, dense_0, E_user, E_item, E_cat,
           W_dense):
    u = user_id.astype(jnp.int32)
    i = item_id.astype(jnp.int32)
    c = category.astype(jnp.int32)
    pu = _pack(E_user, 1000000)
    pi = _pack(E_item, 100000)
    pc = _pack(E_cat, 1000)
    adj = _adjustment(dense_0.T, W_dense)
    return _sc_call(u, i, c, adj, pu, pi, pc)


# speedup vs baseline: 7.3167x; 3.0343x over previous
import functools
import math

import jax
import jax.numpy as jnp
from jax import lax
from jax.experimental import pallas as pl
from jax.experimental.pallas import tpu as pltpu
from jax.experimental.pallas import tpu_sc as plsc

D = 64
B = 16384
SCALE = 1.0 / math.sqrt(4.0)

NC = 2
NS = 16
NW = NC * NS
BPW = B // NW
CH = 128
NCHUNK = BPW // CH


def _adj_body(dt_ref, w_ref, o_ref):
    acc = lax.dot_general(dt_ref[...], w_ref[...], (((0,), (0,)), ((), ())),
                          preferred_element_type=jnp.float32)
    o_ref[...] = acc * SCALE


def _adjustment(dense_t, W_dense):
    return pl.pallas_call(
        _adj_body,
        out_shape=jax.ShapeDtypeStruct((B, D), jnp.float32),
    )(dense_t, W_dense)


def _sc_body(uid_hbm, iid_hbm, cid_hbm, adj_hbm, eu_hbm, ei_hbm, ec_hbm,
             out_hbm, idxu_v, idxi_v, idxc_v, ru_v, ri_v, rc_v,
             adj_v, sem, adj_sem, idx_sem):
    wid = lax.axis_index("s") * NC + lax.axis_index("c")
    base = wid * BPW

    ciu = pltpu.async_copy(uid_hbm.at[pl.ds(base, BPW)], idxu_v, idx_sem)
    cii = pltpu.async_copy(iid_hbm.at[pl.ds(base, BPW)], idxi_v, idx_sem)
    cic = pltpu.async_copy(cid_hbm.at[pl.ds(base, BPW)], idxc_v, idx_sem)
    ciu.wait()
    cii.wait()
    cic.wait()

    def chunk(ci, carry):
        off = base + ci * CH
        loc = ci * CH
        ca = pltpu.async_copy(adj_hbm.at[pl.ds(off, CH)], adj_v, adj_sem)

        def issue(g, rcarry):
            vu = idxu_v[pl.ds(loc + g * 16, 16)]
            vi = idxi_v[pl.ds(loc + g * 16, 16)]
            vc = idxc_v[pl.ds(loc + g * 16, 16)]
            for l in range(16):
                r = g * 16 + l
                pltpu.async_copy(eu_hbm.at[pl.ds(vu[l], 1)],
                                 ru_v.at[pl.ds(r, 1)], sem)
                pltpu.async_copy(ei_hbm.at[pl.ds(vi[l], 1)],
                                 ri_v.at[pl.ds(r, 1)], sem)
                pltpu.async_copy(ec_hbm.at[pl.ds(vc[l], 1)],
                                 rc_v.at[pl.ds(r, 1)], sem)
            return rcarry

        lax.fori_loop(0, CH // 16, issue, 0)
        pltpu.make_async_copy(eu_hbm.at[pl.ds(0, CH)], ru_v, sem).wait()
        pltpu.make_async_copy(ei_hbm.at[pl.ds(0, CH)], ri_v, sem).wait()
        pltpu.make_async_copy(ec_hbm.at[pl.ds(0, CH)], rc_v, sem).wait()
        ca.wait()

        def row(r, rcarry):
            for c4 in range(D // 16):
                s = pl.ds(c4 * 16, 16)
                adj_v[r, s] = (ru_v[r, s] + ri_v[r, s] + rc_v[r, s]) * SCALE \
                    + adj_v[r, s]
            return rcarry

        lax.fori_loop(0, CH, row, 0)
        pltpu.sync_copy(adj_v, out_hbm.at[pl.ds(off, CH)])
        return carry

    lax.fori_loop(0, NCHUNK, chunk, 0)


_sc_call = functools.partial(
    pl.kernel,
    out_type=jax.ShapeDtypeStruct((B, D), jnp.float32),
    mesh=plsc.VectorSubcoreMesh(core_axis_name="c", subcore_axis_name="s"),
    scratch_types=[
        pltpu.VMEM((BPW,), jnp.int32),
        pltpu.VMEM((BPW,), jnp.int32),
        pltpu.VMEM((BPW,), jnp.int32),
        pltpu.VMEM((CH, D), jnp.float32),
        pltpu.VMEM((CH, D), jnp.float32),
        pltpu.VMEM((CH, D), jnp.float32),
        pltpu.VMEM((CH, D), jnp.float32),
        pltpu.SemaphoreType.DMA,
        pltpu.SemaphoreType.DMA,
        pltpu.SemaphoreType.DMA,
    ],
)(_sc_body)


def kernel(user_id, item_id, category, dense_0, E_user, E_item, E_cat,
           W_dense):
    u = user_id.astype(jnp.int32)
    i = item_id.astype(jnp.int32)
    c = category.astype(jnp.int32)
    Eu = E_user.at[0].set(0.0)
    Ei = E_item.at[0].set(0.0)
    Ec = E_cat.at[0].set(0.0)
    adj = _adjustment(dense_0.T, W_dense)
    return _sc_call(u, i, c, adj, Eu, Ei, Ec)
